# baseline (speedup 1.0000x reference)
import functools

import jax
import jax.numpy as jnp
from jax import lax
from jax.experimental import pallas as pl
from jax.experimental.pallas import tpu as pltpu
from jax.experimental.pallas import tpu_sc as plsc

_NC = 2
_NS = 16
_NW = _NC * _NS
_R = 4096
_S = 50
_B = _R * _S
_D = 128
_L = 16
_BPW = _B // _NW
_RPW = _R // _NW
_CR = 8
_NCHUNK = _RPW // _CR

_mesh = plsc.VectorSubcoreMesh(core_axis_name="c", subcore_axis_name="s")

_DNUMS = lax.GatherDimensionNumbers(
    offset_dims=(), collapsed_slice_dims=(0,), start_index_map=(0,))


def _bcast_lane(vec, j):
    idx = jnp.full((_L, 1), j, dtype=jnp.int32)
    return lax.gather(vec, idx, _DNUMS, slice_sizes=(1,),
                      mode=lax.GatherScatterMode.PROMISE_IN_BOUNDS)


@functools.partial(
    pl.kernel,
    out_type=jax.ShapeDtypeStruct((_S, _R, _D), jnp.float32),
    mesh=_mesh,
    compiler_params=pltpu.CompilerParams(use_tc_tiling_on_sc=True),
    scratch_types=[
        pltpu.VMEM((2, _D), jnp.float32),
        pltpu.VMEM((_BPW,), jnp.int32),
        pltpu.VMEM((2, _S, _CR, _D), jnp.float32),
        pltpu.SemaphoreType.DMA((2,)),
    ],
)
def _emb_lookup_sc(idx_hbm, table_hbm, out_hbm,
                   table_v, idx_v, rows_v, sems):
    wid = lax.axis_index("s") * _NC + lax.axis_index("c")
    rbase = wid * _RPW
    pltpu.sync_copy(table_hbm, table_v)
    pltpu.sync_copy(idx_hbm.at[pl.ds(wid * _BPW, _BPW)], idx_v)
    row1 = [table_v[1, pl.ds(k * _L, _L)] for k in range(_D // _L)]
    diff = [table_v[0, pl.ds(k * _L, _L)] - row1[k] for k in range(_D // _L)]
    def compute_chunk(i, buf):
        def body(g, carry):
            q = g * (5 * _L)
            for h in range(5):
                i16 = idx_v[pl.ds(i * (_CR * _S) + q + h * _L, _L)]
                mf = (1 - i16).astype(jnp.float32)
                for j in range(_L):
                    p = q + h * _L + j
                    r = p // _S
                    c = p - r * _S
                    m = _bcast_lane(mf, j)
                    for k in range(_D // _L):
                        buf[c, r, pl.ds(k * _L, _L)] = m * diff[k] + row1[k]
            return carry

        lax.fori_loop(0, _CR * _S // (5 * _L), body, 0)

    def dst(i):
        return out_hbm.at[:, pl.ds(rbase + i * _CR, _CR)]

    def step(i, carry):
        p = i % 2
        buf = rows_v.at[p]
        sem = sems.at[p]

        @pl.when(i >= 2)
        def _drain():
            pltpu.make_async_copy(buf, dst(i), sem).wait()

        compute_chunk(i, buf)
        pltpu.async_copy(buf, dst(i), sem)
        return carry

    lax.fori_loop(0, _NCHUNK, step, 0)
    pltpu.make_async_copy(rows_v.at[0], dst(_NCHUNK - 2), sems.at[0]).wait()
    pltpu.make_async_copy(rows_v.at[1], dst(_NCHUNK - 1), sems.at[1]).wait()


def kernel(inputs, table):
    idx = inputs.reshape(_B)
    out_t = _emb_lookup_sc(idx, table)
    return out_t.transpose(1, 0, 2)


# speedup vs baseline: 1.0374x; 1.0374x over previous
import functools

import jax
import jax.numpy as jnp
from jax import lax
from jax.experimental import pallas as pl
from jax.experimental.pallas import tpu as pltpu
from jax.experimental.pallas import tpu_sc as plsc

_NC = 2
_NS = 16
_NW = _NC * _NS
_R = 4096
_S = 50
_B = _R * _S
_D = 128
_L = 16
_BPW = _B // _NW
_RPW = _R // _NW
_CR = 4
_NCHUNK = _RPW // _CR

_mesh = plsc.VectorSubcoreMesh(core_axis_name="c", subcore_axis_name="s")

_DNUMS = lax.GatherDimensionNumbers(
    offset_dims=(), collapsed_slice_dims=(0,), start_index_map=(0,))


def _bcast_lane(vec, j):
    idx = jnp.full((_L, 1), j, dtype=jnp.int32)
    return lax.gather(vec, idx, _DNUMS, slice_sizes=(1,),
                      mode=lax.GatherScatterMode.PROMISE_IN_BOUNDS)


@functools.partial(
    pl.kernel,
    out_type=jax.ShapeDtypeStruct((_S, _R, _D), jnp.float32),
    mesh=_mesh,
    compiler_params=pltpu.CompilerParams(use_tc_tiling_on_sc=True),
    scratch_types=[
        pltpu.VMEM((2, _D), jnp.float32),
        pltpu.VMEM((_BPW,), jnp.int32),
        pltpu.VMEM((2, _S, _CR, _D), jnp.float32),
        pltpu.SemaphoreType.DMA((2,)),
    ],
)
def _emb_lookup_sc(idx_hbm, table_hbm, out_hbm,
                   table_v, idx_v, rows_v, sems):
    wid = lax.axis_index("s") * _NC + lax.axis_index("c")
    rbase = wid * _RPW
    pltpu.sync_copy(table_hbm, table_v)
    pltpu.sync_copy(idx_hbm.at[pl.ds(wid * _BPW, _BPW)], idx_v)
    row1 = [table_v[1, pl.ds(k * _L, _L)] for k in range(_D // _L)]
    diff = [table_v[0, pl.ds(k * _L, _L)] - row1[k] for k in range(_D // _L)]
    def compute_chunk(i, buf):
        def body(g, carry):
            q = g * _L
            i16 = idx_v[pl.ds(i * (_CR * _S) + q, _L)]
            mf = (1 - i16).astype(jnp.float32)
            for j in range(_L):
                p = q + j
                r = p // _S
                c = p - r * _S
                m = _bcast_lane(mf, j)
                for k in range(_D // _L):
                    buf[c, r, pl.ds(k * _L, _L)] = m * diff[k] + row1[k]
            return carry

        lax.fori_loop(0, _CR * _S // _L, body, 0)

    def dst(i):
        return out_hbm.at[:, pl.ds(rbase + i * _CR, _CR)]

    def step(i, carry):
        p = i % 2
        buf = rows_v.at[p]
        sem = sems.at[p]

        @pl.when(i >= 2)
        def _drain():
            pltpu.make_async_copy(buf, dst(i), sem).wait()

        compute_chunk(i, buf)
        pltpu.async_copy(buf, dst(i), sem)
        return carry

    lax.fori_loop(0, _NCHUNK, step, 0)
    pltpu.make_async_copy(rows_v.at[0], dst(_NCHUNK - 2), sems.at[0]).wait()
    pltpu.make_async_copy(rows_v.at[1], dst(_NCHUNK - 1), sems.at[1]).wait()


def kernel(inputs, table):
    idx = inputs.reshape(_B)
    out_t = _emb_lookup_sc(idx, table)
    return out_t.transpose(1, 0, 2)
